# Initial kernel scaffold; baseline (speedup 1.0000x reference)
import jax
import jax.numpy as jnp
from jax.experimental import pallas as pl


def kernel(h, x_global_features, edge_index, batch_idx, Wl0, bl0, Wr0, Wl1, bl1, Wr1, Wl2, bl2, Wr2, W0, b0, W1, b1, W2, b2, W3, b3):
    raise NotImplementedError("write your pallas kernel here")


# speedup vs baseline: 5.1130x; 5.1130x over previous
import functools

import jax
import jax.numpy as jnp
from jax import lax
from jax.experimental import pallas as pl
from jax.experimental.pallas import tpu as pltpu
from jax.experimental.pallas import tpu_sc as plsc

N = 50000
E = 800000
B = 64
H = 64
NP = 50176
EC = 64
R = 12544
RT = R // 16
K = 16
NCHUNK = RT // K
RPT = NP // 16
BN = 512
GRID = NP // BN

f32 = jnp.float32

_mesh = plsc.VectorSubcoreMesh(core_axis_name="c", subcore_axis_name="s")
_sc_params = pltpu.CompilerParams(use_tc_tiling_on_sc=False)


@functools.partial(
    pl.kernel,
    out_type=jax.ShapeDtypeStruct((NP, 16), f32),
    mesh=_mesh,
    scratch_types=[
        pltpu.VMEM((K, EC), jnp.int32),
        pltpu.VMEM((EC, 16), f32),
        pltpu.VMEM_SHARED((NP, 16), f32),
        pltpu.SemaphoreType.DMA,
    ],
    compiler_params=_sc_params,
)
def _sc_degree(dst_hbm, z16_hbm, ones_hbm, deg_hbm, didx, onesv, dacc, sem):
    c = lax.axis_index("c")
    s = lax.axis_index("s")

    @pl.when(c == 0)
    def _():
        pltpu.sync_copy(z16_hbm.at[pl.ds(s * RPT, RPT)], dacc.at[pl.ds(s * RPT, RPT)])
        pltpu.sync_copy(ones_hbm, onesv)
        plsc.subcore_barrier()

        def chunk(g, carry):
            row0 = s * RT + g * K
            pltpu.sync_copy(dst_hbm.at[pl.ds(row0, K)], didx)
            cps = [pltpu.async_copy(onesv, dacc.at[didx.at[j]], sem, add=True)
                   for j in range(K)]
            for cp in cps:
                cp.wait()
            return carry

        lax.fori_loop(0, NCHUNK, chunk, 0)
        plsc.subcore_barrier()
        pltpu.sync_copy(dacc.at[pl.ds(s * RPT, RPT)], deg_hbm.at[pl.ds(s * RPT, RPT)])


@functools.partial(
    pl.kernel,
    out_type=tuple(jax.ShapeDtypeStruct((NP, 16), f32) for _ in range(4)),
    mesh=_mesh,
    scratch_types=[
        pltpu.VMEM((K, EC), jnp.int32),
        pltpu.VMEM((K, EC), jnp.int32),
        pltpu.VMEM((K, EC, 16), f32),
        pltpu.VMEM_SHARED((NP, 16), f32),
        pltpu.SemaphoreType.DMA,
        pltpu.SemaphoreType.DMA,
    ],
    compiler_params=_sc_params,
)
def _sc_agg(src_hbm, dst_hbm, x0_hbm, x1_hbm, x2_hbm, x3_hbm, z16_hbm,
            ag0_hbm, ag1_hbm, ag2_hbm, ag3_hbm, sidx, didx, rows, acc,
            gsem, ssem):
    c = lax.axis_index("c")
    s = lax.axis_index("s")

    def run(tab_hbm, out_hbm):
        pltpu.sync_copy(z16_hbm.at[pl.ds(s * RPT, RPT)], acc.at[pl.ds(s * RPT, RPT)])
        plsc.subcore_barrier()

        def chunk(g, carry):
            row0 = s * RT + g * K
            pltpu.sync_copy(src_hbm.at[pl.ds(row0, K)], sidx)
            pltpu.sync_copy(dst_hbm.at[pl.ds(row0, K)], didx)
            gs = [pltpu.async_copy(tab_hbm.at[sidx.at[j]], rows.at[j], gsem)
                  for j in range(K)]
            for cp in gs:
                cp.wait()
            ss = [pltpu.async_copy(rows.at[j], acc.at[didx.at[j]], ssem, add=True)
                  for j in range(K)]
            for cp in ss:
                cp.wait()
            return carry

        lax.fori_loop(0, NCHUNK, chunk, 0)
        plsc.subcore_barrier()
        pltpu.sync_copy(acc.at[pl.ds(s * RPT, RPT)], out_hbm.at[pl.ds(s * RPT, RPT)])
        plsc.subcore_barrier()

    @pl.when(c == 0)
    def _():
        run(x0_hbm, ag0_hbm)
        run(x1_hbm, ag1_hbm)

    @pl.when(c == 1)
    def _():
        run(x2_hbm, ag2_hbm)
        run(x3_hbm, ag3_hbm)


def _tc_prep_body(bidx_ref, h_ref, glob_ref, x0_ref, x1_ref, x2_ref, x3_ref):
    oh = (lax.broadcasted_iota(jnp.int32, (BN, B), 1) == bidx_ref[...]).astype(f32)
    gl = jnp.dot(oh, glob_ref[...], preferred_element_type=f32)
    hh = h_ref[...]
    x0_ref[...] = hh[:, 0:16]
    x1_ref[...] = hh[:, 16:32]
    x2_ref[...] = hh[:, 32:48]
    x3_ref[...] = gl


def _tc_prep(bidx2, h_pad, glob):
    return pl.pallas_call(
        _tc_prep_body,
        grid=(GRID,),
        in_specs=[
            pl.BlockSpec((BN, 1), lambda i: (i, 0)),
            pl.BlockSpec((BN, 48), lambda i: (i, 0)),
            pl.BlockSpec((B, 16), lambda i: (0, 0)),
        ],
        out_specs=tuple(pl.BlockSpec((BN, 16), lambda i: (i, 0)) for _ in range(4)),
        out_shape=tuple(jax.ShapeDtypeStruct((NP, 16), f32) for _ in range(4)),
    )(bidx2, h_pad, glob)


def _tc_layer_body(ag0_ref, ag1_ref, ag2_ref, ag3_ref,
                   x0_ref, x1_ref, x2_ref, x3_ref, deg_ref, wc_ref, b_ref,
                   o0_ref, o1_ref, o2_ref, o3_ref):
    r = 1.0 / jnp.maximum(deg_ref[:, 0:1], 1.0)
    a = jnp.concatenate([ag0_ref[...] * r, ag1_ref[...] * r,
                         ag2_ref[...] * r, ag3_ref[...] * r,
                         x0_ref[...], x1_ref[...], x2_ref[...], x3_ref[...]],
                        axis=1)
    y = jnp.dot(a, wc_ref[...], preferred_element_type=f32) + b_ref[...]
    y = jnp.maximum(y, 0.0)
    o0_ref[...] = y[:, 0:16]
    o1_ref[...] = y[:, 16:32]
    o2_ref[...] = y[:, 32:48]
    o3_ref[...] = y[:, 48:64]


def _tc_layer(ags, xs, deg16, wc, bl):
    return pl.pallas_call(
        _tc_layer_body,
        grid=(GRID,),
        in_specs=[pl.BlockSpec((BN, 16), lambda i: (i, 0)) for _ in range(8)]
        + [
            pl.BlockSpec((BN, 16), lambda i: (i, 0)),
            pl.BlockSpec((2 * H, H), lambda i: (0, 0)),
            pl.BlockSpec((1, H), lambda i: (0, 0)),
        ],
        out_specs=tuple(pl.BlockSpec((BN, 16), lambda i: (i, 0)) for _ in range(4)),
        out_shape=tuple(jax.ShapeDtypeStruct((NP, 16), f32) for _ in range(4)),
    )(*ags, *xs, deg16, wc, bl)


def _tc_final_body(ag0_ref, ag1_ref, ag2_ref, ag3_ref,
                   x0_ref, x1_ref, x2_ref, x3_ref, deg_ref, bidx_ref,
                   wc_ref, b_ref, w0_ref, b0_ref, w1_ref, b1_ref,
                   w2_ref, b2_ref, w3_ref, b3_ref, out_ref, gsum, gcnt):
    i = pl.program_id(0)

    @pl.when(i == 0)
    def _():
        gsum[...] = jnp.zeros((B, H), f32)
        gcnt[...] = jnp.zeros((B, H), f32)

    r = 1.0 / jnp.maximum(deg_ref[:, 0:1], 1.0)
    a = jnp.concatenate([ag0_ref[...] * r, ag1_ref[...] * r,
                         ag2_ref[...] * r, ag3_ref[...] * r,
                         x0_ref[...], x1_ref[...], x2_ref[...], x3_ref[...]],
                        axis=1)
    y = jnp.dot(a, wc_ref[...], preferred_element_type=f32) + b_ref[...]
    oh = (lax.broadcasted_iota(jnp.int32, (BN, B), 1) == bidx_ref[...]).astype(f32)
    gsum[...] += lax.dot_general(oh, y, (((0,), (0,)), ((), ())),
                                 preferred_element_type=f32)
    gcnt[...] += lax.dot_general(oh, jnp.ones((BN, H), f32),
                                 (((0,), (0,)), ((), ())),
                                 preferred_element_type=f32)

    @pl.when(i == GRID - 1)
    def _():
        g = gsum[...] / jnp.maximum(gcnt[...], 1.0)
        g = jnp.maximum(jnp.dot(g, w0_ref[...], preferred_element_type=f32)
                        + b0_ref[...], 0.0)
        g = jnp.maximum(jnp.dot(g, w1_ref[...], preferred_element_type=f32)
                        + b1_ref[...], 0.0)
        g = jnp.maximum(jnp.dot(g, w2_ref[...], preferred_element_type=f32)
                        + b2_ref[...], 0.0)
        out_ref[...] = jnp.dot(g, w3_ref[...], preferred_element_type=f32) + b3_ref[...]


def _tc_final(ags, xs, deg16, bidx2, wc, bl,
              w0t, b0r, w1t, b1r, w2t, b2r, w3t, b3r):
    return pl.pallas_call(
        _tc_final_body,
        grid=(GRID,),
        in_specs=[pl.BlockSpec((BN, 16), lambda i: (i, 0)) for _ in range(8)]
        + [
            pl.BlockSpec((BN, 16), lambda i: (i, 0)),
            pl.BlockSpec((BN, 1), lambda i: (i, 0)),
            pl.BlockSpec((2 * H, H), lambda i: (0, 0)),
            pl.BlockSpec((1, H), lambda i: (0, 0)),
            pl.BlockSpec((H, H), lambda i: (0, 0)),
            pl.BlockSpec((1, H), lambda i: (0, 0)),
            pl.BlockSpec((H, H), lambda i: (0, 0)),
            pl.BlockSpec((1, H), lambda i: (0, 0)),
            pl.BlockSpec((H, H), lambda i: (0, 0)),
            pl.BlockSpec((1, H), lambda i: (0, 0)),
            pl.BlockSpec((H, 1), lambda i: (0, 0)),
            pl.BlockSpec((1, 1), lambda i: (0, 0)),
        ],
        out_specs=pl.BlockSpec((B, 1), lambda i: (0, 0)),
        out_shape=jax.ShapeDtypeStruct((B, 1), f32),
        scratch_shapes=[pltpu.VMEM((B, H), f32), pltpu.VMEM((B, H), f32)],
    )(*ags, *xs, deg16, bidx2, wc, bl,
      w0t, b0r, w1t, b1r, w2t, b2r, w3t, b3r)


def kernel(h, x_global_features, edge_index, batch_idx,
           Wl0, bl0, Wr0, Wl1, bl1, Wr1, Wl2, bl2, Wr2,
           W0, b0, W1, b1, W2, b2, W3, b3):
    i32 = jnp.int32
    pad_e = R * EC - E
    src_r = jnp.concatenate([edge_index[0], jnp.full((pad_e,), N, i32)]).reshape(R, EC)
    dst_r = jnp.concatenate([edge_index[1], jnp.full((pad_e,), N, i32)]).reshape(R, EC)
    h_pad = jnp.pad(h, ((0, NP - N), (0, 0)))
    bidx2 = jnp.pad(batch_idx, (0, NP - N), constant_values=B).reshape(NP, 1)
    z16 = jnp.zeros((NP, 16), f32)
    ones_e = jnp.ones((EC, 16), f32)

    deg16 = _sc_degree(dst_r, z16, ones_e)
    xs = _tc_prep(bidx2, h_pad, x_global_features)

    wcs = [jnp.concatenate([Wl0.T, Wr0.T], axis=0),
           jnp.concatenate([Wl1.T, Wr1.T], axis=0),
           jnp.concatenate([Wl2.T, Wr2.T], axis=0)]
    bls = [bl0.reshape(1, H), bl1.reshape(1, H), bl2.reshape(1, H)]

    for layer in range(2):
        ags = _sc_agg(src_r, dst_r, *xs, z16)
        xs = _tc_layer(ags, xs, deg16, wcs[layer], bls[layer])

    ags = _sc_agg(src_r, dst_r, *xs, z16)
    out = _tc_final(ags, xs, deg16, bidx2, wcs[2], bls[2],
                    W0.T, b0.reshape(1, H), W1.T, b1.reshape(1, H),
                    W2.T, b2.reshape(1, H), W3.T, b3.reshape(1, 1))
    return out.reshape(B)


# speedup vs baseline: 6.1478x; 1.2024x over previous
import functools

import jax
import jax.numpy as jnp
from jax import lax
from jax.experimental import pallas as pl
from jax.experimental.pallas import tpu as pltpu
from jax.experimental.pallas import tpu_sc as plsc

N = 50000
E = 800000
B = 64
H = 64
NP = 50176
EC = 64
R = 12544
RT = R // 16
K = 14
NCHUNK = RT // K
NG = NCHUNK // 2
RT2 = RT // 2
NCHUNK2 = RT2 // K
RPT = NP // 16
BN = 512
GRID = NP // BN

f32 = jnp.float32

_mesh = plsc.VectorSubcoreMesh(core_axis_name="c", subcore_axis_name="s")
_sc_params = pltpu.CompilerParams(use_tc_tiling_on_sc=False)


@functools.partial(
    pl.kernel,
    out_type=(jax.ShapeDtypeStruct((NP, 16), f32),
              jax.ShapeDtypeStruct((NP, 16), f32)),
    mesh=_mesh,
    scratch_types=[
        pltpu.VMEM((K, EC), jnp.int32),
        pltpu.VMEM((EC, 16), f32),
        pltpu.VMEM_SHARED((NP, 16), f32),
        pltpu.SemaphoreType.DMA,
    ],
    compiler_params=_sc_params,
)
def _sc_degree(dst_hbm, z16_hbm, ones_hbm, dega_hbm, degb_hbm,
               didx, onesv, dacc, sem):
    c = lax.axis_index("c")
    s = lax.axis_index("s")

    def run(base, out_hbm):
        pltpu.sync_copy(z16_hbm.at[pl.ds(s * RPT, RPT)], dacc.at[pl.ds(s * RPT, RPT)])
        pltpu.sync_copy(ones_hbm, onesv)
        plsc.subcore_barrier()

        def chunk(g, carry):
            row0 = base + s * RT2 + g * K
            pltpu.sync_copy(dst_hbm.at[pl.ds(row0, K)], didx)
            cps = [pltpu.async_copy(onesv, dacc.at[didx.at[j]], sem, add=True)
                   for j in range(K)]
            for cp in cps:
                cp.wait()
            return carry

        lax.fori_loop(0, NCHUNK2, chunk, 0)
        plsc.subcore_barrier()
        pltpu.sync_copy(dacc.at[pl.ds(s * RPT, RPT)], out_hbm.at[pl.ds(s * RPT, RPT)])

    @pl.when(c == 0)
    def _():
        run(0, dega_hbm)

    @pl.when(c == 1)
    def _():
        run(R // 2, degb_hbm)


@functools.partial(
    pl.kernel,
    out_type=tuple(jax.ShapeDtypeStruct((NP, 16), f32) for _ in range(4)),
    mesh=_mesh,
    scratch_types=[
        pltpu.VMEM((2 * K, EC), jnp.int32),
        pltpu.VMEM((2 * K, EC), jnp.int32),
        pltpu.VMEM((2 * K * EC, 16), f32),
        pltpu.VMEM_SHARED((NP, 16), f32),
        pltpu.SemaphoreType.DMA,
        pltpu.SemaphoreType.DMA,
        pltpu.SemaphoreType.DMA,
        pltpu.SemaphoreType.DMA,
    ],
    compiler_params=_sc_params,
)
def _sc_agg(src_hbm, dst_hbm, x0_hbm, x1_hbm, x2_hbm, x3_hbm, z16_hbm,
            ag0_hbm, ag1_hbm, ag2_hbm, ag3_hbm, sidx, didx, rows, acc,
            gsem0, gsem1, ssem0, ssem1):
    c = lax.axis_index("c")
    s = lax.axis_index("s")
    gsems = (gsem0, gsem1)
    ssems = (ssem0, ssem1)
    drain_bytes_rows = K * EC

    def run(tab_hbm, out_hbm):
        pltpu.sync_copy(z16_hbm.at[pl.ds(s * RPT, RPT)], acc.at[pl.ds(s * RPT, RPT)])
        plsc.subcore_barrier()

        def load_idx(ch, p):
            row0 = s * RT + ch * K
            pltpu.sync_copy(src_hbm.at[pl.ds(row0, K)], sidx.at[pl.ds(p * K, K)])
            pltpu.sync_copy(dst_hbm.at[pl.ds(row0, K)], didx.at[pl.ds(p * K, K)])

        def fire_gather(p):
            for j in range(K):
                pltpu.async_copy(tab_hbm.at[sidx.at[p * K + j]],
                                 rows.at[pl.ds((p * K + j) * EC, EC)], gsems[p])

        def fire_scatter(p):
            for j in range(K):
                pltpu.async_copy(rows.at[pl.ds((p * K + j) * EC, EC)],
                                 acc.at[didx.at[p * K + j]], ssems[p], add=True)

        def drain(p, sems):
            pltpu.make_async_copy(
                z16_hbm.at[pl.ds(0, drain_bytes_rows)],
                rows.at[pl.ds(p * K * EC, drain_bytes_rows)], sems[p]).wait()

        load_idx(0, 0)
        fire_gather(0)

        def body2(gg, carry):
            @pl.when(gg >= 1)
            def _():
                drain(1, ssems)
            load_idx(2 * gg + 1, 1)
            fire_gather(1)
            drain(0, gsems)
            fire_scatter(0)

            @pl.when(gg < NG - 1)
            def _():
                load_idx(2 * gg + 2, 0)
                drain(0, ssems)
                fire_gather(0)

            @pl.when(gg == NG - 1)
            def _():
                drain(0, ssems)

            drain(1, gsems)
            fire_scatter(1)
            return carry

        lax.fori_loop(0, NG, body2, 0)
        drain(1, ssems)
        plsc.subcore_barrier()
        pltpu.sync_copy(acc.at[pl.ds(s * RPT, RPT)], out_hbm.at[pl.ds(s * RPT, RPT)])
        plsc.subcore_barrier()

    @pl.when(c == 0)
    def _():
        run(x0_hbm, ag0_hbm)
        run(x1_hbm, ag1_hbm)

    @pl.when(c == 1)
    def _():
        run(x2_hbm, ag2_hbm)
        run(x3_hbm, ag3_hbm)


def _tc_prep_body(bidx_ref, h_ref, glob_ref, dega_ref, degb_ref,
                  x0_ref, x1_ref, x2_ref, x3_ref, rcp_ref):
    oh = (lax.broadcasted_iota(jnp.int32, (BN, B), 1) == bidx_ref[...]).astype(f32)
    gl = jnp.dot(oh, glob_ref[...], preferred_element_type=f32, precision=lax.Precision.HIGHEST)
    hh = h_ref[...]
    x0_ref[...] = hh[:, 0:16]
    x1_ref[...] = hh[:, 16:32]
    x2_ref[...] = hh[:, 32:48]
    x3_ref[...] = gl
    rcp_ref[...] = jnp.maximum(dega_ref[...] + degb_ref[...], 1.0)


def _tc_prep(bidx2, h_pad, glob, dega, degb):
    return pl.pallas_call(
        _tc_prep_body,
        grid=(GRID,),
        in_specs=[
            pl.BlockSpec((BN, 1), lambda i: (i, 0)),
            pl.BlockSpec((BN, 48), lambda i: (i, 0)),
            pl.BlockSpec((B, 16), lambda i: (0, 0)),
            pl.BlockSpec((BN, 16), lambda i: (i, 0)),
            pl.BlockSpec((BN, 16), lambda i: (i, 0)),
        ],
        out_specs=tuple(pl.BlockSpec((BN, 16), lambda i: (i, 0)) for _ in range(5)),
        out_shape=tuple(jax.ShapeDtypeStruct((NP, 16), f32) for _ in range(5)),
    )(bidx2, h_pad, glob, dega, degb)


def _tc_layer_body(ag0_ref, ag1_ref, ag2_ref, ag3_ref,
                   x0_ref, x1_ref, x2_ref, x3_ref, deg_ref, wl_ref, b_ref,
                   wr_ref, o0_ref, o1_ref, o2_ref, o3_ref):
    d = deg_ref[:, 0:1]
    mean = jnp.concatenate([ag0_ref[...] / d, ag1_ref[...] / d,
                            ag2_ref[...] / d, ag3_ref[...] / d], axis=1)
    x = jnp.concatenate([x0_ref[...], x1_ref[...], x2_ref[...], x3_ref[...]],
                        axis=1)
    y = (jnp.dot(mean, wl_ref[...], preferred_element_type=f32) + b_ref[...]
         + jnp.dot(x, wr_ref[...], preferred_element_type=f32))
    y = jnp.maximum(y, 0.0)
    o0_ref[...] = y[:, 0:16]
    o1_ref[...] = y[:, 16:32]
    o2_ref[...] = y[:, 32:48]
    o3_ref[...] = y[:, 48:64]


def _tc_layer(ags, xs, deg16, wlt, bl, wrt):
    return pl.pallas_call(
        _tc_layer_body,
        grid=(GRID,),
        in_specs=[pl.BlockSpec((BN, 16), lambda i: (i, 0)) for _ in range(8)]
        + [
            pl.BlockSpec((BN, 16), lambda i: (i, 0)),
            pl.BlockSpec((H, H), lambda i: (0, 0)),
            pl.BlockSpec((1, H), lambda i: (0, 0)),
            pl.BlockSpec((H, H), lambda i: (0, 0)),
        ],
        out_specs=tuple(pl.BlockSpec((BN, 16), lambda i: (i, 0)) for _ in range(4)),
        out_shape=tuple(jax.ShapeDtypeStruct((NP, 16), f32) for _ in range(4)),
    )(*ags, *xs, deg16, wlt, bl, wrt)


def _tc_final_body(ag0_ref, ag1_ref, ag2_ref, ag3_ref,
                   x0_ref, x1_ref, x2_ref, x3_ref, deg_ref, bidx_ref,
                   wl_ref, b_ref, wr_ref, w0_ref, b0_ref, w1_ref, b1_ref,
                   w2_ref, b2_ref, w3_ref, b3_ref, out_ref, gsum, gcnt, gcomp):
    i = pl.program_id(0)

    @pl.when(i == 0)
    def _():
        gsum[...] = jnp.zeros((B, H), f32)
        gcnt[...] = jnp.zeros((B, H), f32)
        gcomp[...] = jnp.zeros((B, H), f32)

    d = deg_ref[:, 0:1]
    mean = jnp.concatenate([ag0_ref[...] / d, ag1_ref[...] / d,
                            ag2_ref[...] / d, ag3_ref[...] / d], axis=1)
    x = jnp.concatenate([x0_ref[...], x1_ref[...], x2_ref[...], x3_ref[...]],
                        axis=1)
    y = (jnp.dot(mean, wl_ref[...], preferred_element_type=f32) + b_ref[...]
         + jnp.dot(x, wr_ref[...], preferred_element_type=f32))
    oh = (lax.broadcasted_iota(jnp.int32, (BN, B), 1) == bidx_ref[...]).astype(f32)
    blk = lax.dot_general(oh, y, (((0,), (0,)), ((), ())),
                          preferred_element_type=f32,
                          precision=lax.Precision.HIGHEST)
    yk = blk - gcomp[...]
    t = gsum[...] + yk
    gcomp[...] = (t - gsum[...]) - yk
    gsum[...] = t
    gcnt[...] += lax.dot_general(oh, jnp.ones((BN, H), f32),
                                 (((0,), (0,)), ((), ())),
                                 preferred_element_type=f32,
                                 precision=lax.Precision.HIGHEST)

    @pl.when(i == GRID - 1)
    def _():
        g = gsum[...] / jnp.maximum(gcnt[...], 1.0)
        g = jnp.maximum(jnp.dot(g, w0_ref[...], preferred_element_type=f32)
                        + b0_ref[...], 0.0)
        g = jnp.maximum(jnp.dot(g, w1_ref[...], preferred_element_type=f32)
                        + b1_ref[...], 0.0)
        g = jnp.maximum(jnp.dot(g, w2_ref[...], preferred_element_type=f32)
                        + b2_ref[...], 0.0)
        out_ref[...] = jnp.dot(g, w3_ref[...], preferred_element_type=f32) + b3_ref[...]


def _tc_final(ags, xs, deg16, bidx2, wlt, bl, wrt,
              w0t, b0r, w1t, b1r, w2t, b2r, w3t, b3r):
    return pl.pallas_call(
        _tc_final_body,
        grid=(GRID,),
        in_specs=[pl.BlockSpec((BN, 16), lambda i: (i, 0)) for _ in range(8)]
        + [
            pl.BlockSpec((BN, 16), lambda i: (i, 0)),
            pl.BlockSpec((BN, 1), lambda i: (i, 0)),
            pl.BlockSpec((H, H), lambda i: (0, 0)),
            pl.BlockSpec((1, H), lambda i: (0, 0)),
            pl.BlockSpec((H, H), lambda i: (0, 0)),
            pl.BlockSpec((H, H), lambda i: (0, 0)),
            pl.BlockSpec((1, H), lambda i: (0, 0)),
            pl.BlockSpec((H, H), lambda i: (0, 0)),
            pl.BlockSpec((1, H), lambda i: (0, 0)),
            pl.BlockSpec((H, H), lambda i: (0, 0)),
            pl.BlockSpec((1, H), lambda i: (0, 0)),
            pl.BlockSpec((H, 1), lambda i: (0, 0)),
            pl.BlockSpec((1, 1), lambda i: (0, 0)),
        ],
        out_specs=pl.BlockSpec((B, 1), lambda i: (0, 0)),
        out_shape=jax.ShapeDtypeStruct((B, 1), f32),
        scratch_shapes=[pltpu.VMEM((B, H), f32), pltpu.VMEM((B, H), f32),
                        pltpu.VMEM((B, H), f32)],
    )(*ags, *xs, deg16, bidx2, wlt, bl, wrt,
      w0t, b0r, w1t, b1r, w2t, b2r, w3t, b3r)


def kernel(h, x_global_features, edge_index, batch_idx,
           Wl0, bl0, Wr0, Wl1, bl1, Wr1, Wl2, bl2, Wr2,
           W0, b0, W1, b1, W2, b2, W3, b3):
    i32 = jnp.int32
    pad_e = R * EC - E
    src_r = jnp.concatenate([edge_index[0], jnp.full((pad_e,), N, i32)]).reshape(R, EC)
    dst_r = jnp.concatenate([edge_index[1], jnp.full((pad_e,), N, i32)]).reshape(R, EC)
    h_pad = jnp.pad(h, ((0, NP - N), (0, 0)))
    bidx2 = jnp.pad(batch_idx, (0, NP - N), constant_values=B).reshape(NP, 1)
    z16 = jnp.zeros((NP, 16), f32)
    ones_e = jnp.ones((EC, 16), f32)

    dega, degb = _sc_degree(dst_r, z16, ones_e)
    *xs, deg16 = _tc_prep(bidx2, h_pad, x_global_features, dega, degb)

    wls = [Wl0.T, Wl1.T, Wl2.T]
    wrs = [Wr0.T, Wr1.T, Wr2.T]
    bls = [bl0.reshape(1, H), bl1.reshape(1, H), bl2.reshape(1, H)]

    for layer in range(2):
        ags = _sc_agg(src_r, dst_r, *xs, z16)
        xs = list(_tc_layer(ags, xs, deg16, wls[layer], bls[layer], wrs[layer]))

    ags = _sc_agg(src_r, dst_r, *xs, z16)
    out = _tc_final(ags, xs, deg16, bidx2, wls[2], bls[2], wrs[2],
                    W0.T, b0.reshape(1, H), W1.T, b1.reshape(1, H),
                    W2.T, b2.reshape(1, H), W3.T, b3.reshape(1, 1))
    return out.reshape(B)


# speedup vs baseline: 6.7795x; 1.1028x over previous
import functools

import jax
import jax.numpy as jnp
from jax import lax
from jax.experimental import pallas as pl
from jax.experimental.pallas import tpu as pltpu
from jax.experimental.pallas import tpu_sc as plsc

N = 50000
E = 800000
B = 64
H = 64
NP = 50176
EC = 64
R = 12544
RT = R // 16
K = 14
NCHUNK = RT // K
NG = NCHUNK // 2
RT2 = RT // 2
NCHUNK2 = RT2 // K
RPT = NP // 16
BN = 3136
GRID = NP // BN

f32 = jnp.float32

_mesh = plsc.VectorSubcoreMesh(core_axis_name="c", subcore_axis_name="s")
_sc_params = pltpu.CompilerParams(use_tc_tiling_on_sc=False)


@functools.partial(
    pl.kernel,
    out_type=(jax.ShapeDtypeStruct((NP, 16), f32),
              jax.ShapeDtypeStruct((NP, 16), f32)),
    mesh=_mesh,
    scratch_types=[
        pltpu.VMEM((K, EC), jnp.int32),
        pltpu.VMEM((EC, 16), f32),
        pltpu.VMEM_SHARED((NP, 16), f32),
        pltpu.SemaphoreType.DMA,
    ],
    compiler_params=_sc_params,
)
def _sc_degree(dst_hbm, z16_hbm, ones_hbm, dega_hbm, degb_hbm,
               didx, onesv, dacc, sem):
    c = lax.axis_index("c")
    s = lax.axis_index("s")

    def run(base, out_hbm):
        pltpu.sync_copy(z16_hbm.at[pl.ds(s * RPT, RPT)], dacc.at[pl.ds(s * RPT, RPT)])
        pltpu.sync_copy(ones_hbm, onesv)
        plsc.subcore_barrier()

        def chunk(g, carry):
            row0 = base + s * RT2 + g * K
            pltpu.sync_copy(dst_hbm.at[pl.ds(row0, K)], didx)
            cps = [pltpu.async_copy(onesv, dacc.at[didx.at[j]], sem, add=True)
                   for j in range(K)]
            for cp in cps:
                cp.wait()
            return carry

        lax.fori_loop(0, NCHUNK2, chunk, 0)
        plsc.subcore_barrier()
        pltpu.sync_copy(dacc.at[pl.ds(s * RPT, RPT)], out_hbm.at[pl.ds(s * RPT, RPT)])

    @pl.when(c == 0)
    def _():
        run(0, dega_hbm)

    @pl.when(c == 1)
    def _():
        run(R // 2, degb_hbm)


@functools.partial(
    pl.kernel,
    out_type=tuple(jax.ShapeDtypeStruct((NP, 16), f32) for _ in range(4)),
    mesh=_mesh,
    scratch_types=[
        pltpu.VMEM((2 * K, EC), jnp.int32),
        pltpu.VMEM((2 * K, EC), jnp.int32),
        pltpu.VMEM((2 * K * EC, 16), f32),
        pltpu.VMEM_SHARED((NP, 16), f32),
        pltpu.SemaphoreType.DMA,
        pltpu.SemaphoreType.DMA,
        pltpu.SemaphoreType.DMA,
        pltpu.SemaphoreType.DMA,
    ],
    compiler_params=_sc_params,
)
def _sc_agg(src_hbm, dst_hbm, x0_hbm, x1_hbm, x2_hbm, x3_hbm, z16_hbm,
            ag0_hbm, ag1_hbm, ag2_hbm, ag3_hbm, sidx, didx, rows, acc,
            gsem0, gsem1, ssem0, ssem1):
    c = lax.axis_index("c")
    s = lax.axis_index("s")
    gsems = (gsem0, gsem1)
    ssems = (ssem0, ssem1)
    drain_bytes_rows = K * EC

    def run(tab_hbm, out_hbm):
        pltpu.sync_copy(z16_hbm.at[pl.ds(s * RPT, RPT)], acc.at[pl.ds(s * RPT, RPT)])
        plsc.subcore_barrier()

        def load_idx(ch, p):
            row0 = s * RT + ch * K
            pltpu.sync_copy(src_hbm.at[pl.ds(row0, K)], sidx.at[pl.ds(p * K, K)])
            pltpu.sync_copy(dst_hbm.at[pl.ds(row0, K)], didx.at[pl.ds(p * K, K)])

        def fire_gather(p):
            for j in range(K):
                pltpu.async_copy(tab_hbm.at[sidx.at[p * K + j]],
                                 rows.at[pl.ds((p * K + j) * EC, EC)], gsems[p])

        def fire_scatter(p):
            for j in range(K):
                pltpu.async_copy(rows.at[pl.ds((p * K + j) * EC, EC)],
                                 acc.at[didx.at[p * K + j]], ssems[p], add=True)

        def drain(p, sems):
            pltpu.make_async_copy(
                z16_hbm.at[pl.ds(0, drain_bytes_rows)],
                rows.at[pl.ds(p * K * EC, drain_bytes_rows)], sems[p]).wait()

        load_idx(0, 0)
        fire_gather(0)

        def body2(gg, carry):
            @pl.when(gg >= 1)
            def _():
                drain(1, ssems)
            load_idx(2 * gg + 1, 1)
            fire_gather(1)
            drain(0, gsems)
            fire_scatter(0)

            @pl.when(gg < NG - 1)
            def _():
                load_idx(2 * gg + 2, 0)
                drain(0, ssems)
                fire_gather(0)

            @pl.when(gg == NG - 1)
            def _():
                drain(0, ssems)

            drain(1, gsems)
            fire_scatter(1)
            return carry

        lax.fori_loop(0, NG, body2, 0)
        drain(1, ssems)
        plsc.subcore_barrier()
        pltpu.sync_copy(acc.at[pl.ds(s * RPT, RPT)], out_hbm.at[pl.ds(s * RPT, RPT)])
        plsc.subcore_barrier()

    @pl.when(c == 0)
    def _():
        run(x0_hbm, ag0_hbm)
        run(x1_hbm, ag1_hbm)

    @pl.when(c == 1)
    def _():
        run(x2_hbm, ag2_hbm)
        run(x3_hbm, ag3_hbm)


def _tc_prep_body(bidx_ref, h_ref, glob_ref, dega_ref, degb_ref,
                  x0_ref, x1_ref, x2_ref, x3_ref, rcp_ref):
    oh = (lax.broadcasted_iota(jnp.int32, (BN, B), 1) == bidx_ref[...]).astype(f32)
    gl = jnp.dot(oh, glob_ref[...], preferred_element_type=f32, precision=lax.Precision.HIGHEST)
    hh = h_ref[...]
    x0_ref[...] = hh[:, 0:16]
    x1_ref[...] = hh[:, 16:32]
    x2_ref[...] = hh[:, 32:48]
    x3_ref[...] = gl
    rcp_ref[...] = jnp.maximum(dega_ref[...] + degb_ref[...], 1.0)


def _tc_prep(bidx2, h_pad, glob, dega, degb):
    return pl.pallas_call(
        _tc_prep_body,
        grid=(GRID,),
        in_specs=[
            pl.BlockSpec((BN, 1), lambda i: (i, 0)),
            pl.BlockSpec((BN, 48), lambda i: (i, 0)),
            pl.BlockSpec((B, 16), lambda i: (0, 0)),
            pl.BlockSpec((BN, 16), lambda i: (i, 0)),
            pl.BlockSpec((BN, 16), lambda i: (i, 0)),
        ],
        out_specs=tuple(pl.BlockSpec((BN, 16), lambda i: (i, 0)) for _ in range(5)),
        out_shape=tuple(jax.ShapeDtypeStruct((NP, 16), f32) for _ in range(5)),
    )(bidx2, h_pad, glob, dega, degb)


def _tc_layer_body(ag0_ref, ag1_ref, ag2_ref, ag3_ref,
                   x0_ref, x1_ref, x2_ref, x3_ref, deg_ref, wl_ref, b_ref,
                   wr_ref, o0_ref, o1_ref, o2_ref, o3_ref):
    d = deg_ref[:, 0:1]
    mean = jnp.concatenate([ag0_ref[...] / d, ag1_ref[...] / d,
                            ag2_ref[...] / d, ag3_ref[...] / d], axis=1)
    x = jnp.concatenate([x0_ref[...], x1_ref[...], x2_ref[...], x3_ref[...]],
                        axis=1)
    y = (jnp.dot(mean, wl_ref[...], preferred_element_type=f32) + b_ref[...]
         + jnp.dot(x, wr_ref[...], preferred_element_type=f32))
    y = jnp.maximum(y, 0.0)
    o0_ref[...] = y[:, 0:16]
    o1_ref[...] = y[:, 16:32]
    o2_ref[...] = y[:, 32:48]
    o3_ref[...] = y[:, 48:64]


def _tc_layer(ags, xs, deg16, wlt, bl, wrt):
    return pl.pallas_call(
        _tc_layer_body,
        grid=(GRID,),
        in_specs=[pl.BlockSpec((BN, 16), lambda i: (i, 0)) for _ in range(8)]
        + [
            pl.BlockSpec((BN, 16), lambda i: (i, 0)),
            pl.BlockSpec((H, H), lambda i: (0, 0)),
            pl.BlockSpec((1, H), lambda i: (0, 0)),
            pl.BlockSpec((H, H), lambda i: (0, 0)),
        ],
        out_specs=tuple(pl.BlockSpec((BN, 16), lambda i: (i, 0)) for _ in range(4)),
        out_shape=tuple(jax.ShapeDtypeStruct((NP, 16), f32) for _ in range(4)),
    )(*ags, *xs, deg16, wlt, bl, wrt)


def _tc_final_body(ag0_ref, ag1_ref, ag2_ref, ag3_ref,
                   x0_ref, x1_ref, x2_ref, x3_ref, deg_ref, bidx_ref,
                   wl_ref, b_ref, wr_ref, w0_ref, b0_ref, w1_ref, b1_ref,
                   w2_ref, b2_ref, w3_ref, b3_ref, out_ref, gsum, gcnt, gcomp):
    i = pl.program_id(0)

    @pl.when(i == 0)
    def _():
        gsum[...] = jnp.zeros((B, H), f32)
        gcnt[...] = jnp.zeros((B, H), f32)
        gcomp[...] = jnp.zeros((B, H), f32)

    d = deg_ref[:, 0:1]
    mean = jnp.concatenate([ag0_ref[...] / d, ag1_ref[...] / d,
                            ag2_ref[...] / d, ag3_ref[...] / d], axis=1)
    x = jnp.concatenate([x0_ref[...], x1_ref[...], x2_ref[...], x3_ref[...]],
                        axis=1)
    y = (jnp.dot(mean, wl_ref[...], preferred_element_type=f32) + b_ref[...]
         + jnp.dot(x, wr_ref[...], preferred_element_type=f32))
    oh = (lax.broadcasted_iota(jnp.int32, (BN, B), 1) == bidx_ref[...]).astype(f32)
    blk = lax.dot_general(oh, y, (((0,), (0,)), ((), ())),
                          preferred_element_type=f32,
                          precision=lax.Precision.HIGHEST)
    yk = blk - gcomp[...]
    t = gsum[...] + yk
    gcomp[...] = (t - gsum[...]) - yk
    gsum[...] = t
    gcnt[...] += lax.dot_general(oh, jnp.ones((BN, H), f32),
                                 (((0,), (0,)), ((), ())),
                                 preferred_element_type=f32,
                                 precision=lax.Precision.HIGHEST)

    @pl.when(i == GRID - 1)
    def _():
        g = gsum[...] / jnp.maximum(gcnt[...], 1.0)
        g = jnp.maximum(jnp.dot(g, w0_ref[...], preferred_element_type=f32)
                        + b0_ref[...], 0.0)
        g = jnp.maximum(jnp.dot(g, w1_ref[...], preferred_element_type=f32)
                        + b1_ref[...], 0.0)
        g = jnp.maximum(jnp.dot(g, w2_ref[...], preferred_element_type=f32)
                        + b2_ref[...], 0.0)
        out_ref[...] = jnp.dot(g, w3_ref[...], preferred_element_type=f32) + b3_ref[...]


def _tc_final(ags, xs, deg16, bidx2, wlt, bl, wrt,
              w0t, b0r, w1t, b1r, w2t, b2r, w3t, b3r):
    return pl.pallas_call(
        _tc_final_body,
        grid=(GRID,),
        in_specs=[pl.BlockSpec((BN, 16), lambda i: (i, 0)) for _ in range(8)]
        + [
            pl.BlockSpec((BN, 16), lambda i: (i, 0)),
            pl.BlockSpec((BN, 1), lambda i: (i, 0)),
            pl.BlockSpec((H, H), lambda i: (0, 0)),
            pl.BlockSpec((1, H), lambda i: (0, 0)),
            pl.BlockSpec((H, H), lambda i: (0, 0)),
            pl.BlockSpec((H, H), lambda i: (0, 0)),
            pl.BlockSpec((1, H), lambda i: (0, 0)),
            pl.BlockSpec((H, H), lambda i: (0, 0)),
            pl.BlockSpec((1, H), lambda i: (0, 0)),
            pl.BlockSpec((H, H), lambda i: (0, 0)),
            pl.BlockSpec((1, H), lambda i: (0, 0)),
            pl.BlockSpec((H, 1), lambda i: (0, 0)),
            pl.BlockSpec((1, 1), lambda i: (0, 0)),
        ],
        out_specs=pl.BlockSpec((B, 1), lambda i: (0, 0)),
        out_shape=jax.ShapeDtypeStruct((B, 1), f32),
        scratch_shapes=[pltpu.VMEM((B, H), f32), pltpu.VMEM((B, H), f32),
                        pltpu.VMEM((B, H), f32)],
    )(*ags, *xs, deg16, bidx2, wlt, bl, wrt,
      w0t, b0r, w1t, b1r, w2t, b2r, w3t, b3r)


def kernel(h, x_global_features, edge_index, batch_idx,
           Wl0, bl0, Wr0, Wl1, bl1, Wr1, Wl2, bl2, Wr2,
           W0, b0, W1, b1, W2, b2, W3, b3):
    i32 = jnp.int32
    pad_e = R * EC - E
    src_r = jnp.concatenate([edge_index[0], jnp.full((pad_e,), N, i32)]).reshape(R, EC)
    dst_r = jnp.concatenate([edge_index[1], jnp.full((pad_e,), N, i32)]).reshape(R, EC)
    h_pad = jnp.pad(h, ((0, NP - N), (0, 0)))
    bidx2 = jnp.pad(batch_idx, (0, NP - N), constant_values=B).reshape(NP, 1)
    z16 = jnp.zeros((NP, 16), f32)
    ones_e = jnp.ones((EC, 16), f32)

    dega, degb = _sc_degree(dst_r, z16, ones_e)
    *xs, deg16 = _tc_prep(bidx2, h_pad, x_global_features, dega, degb)

    wls = [Wl0.T, Wl1.T, Wl2.T]
    wrs = [Wr0.T, Wr1.T, Wr2.T]
    bls = [bl0.reshape(1, H), bl1.reshape(1, H), bl2.reshape(1, H)]

    for layer in range(2):
        ags = _sc_agg(src_r, dst_r, *xs, z16)
        xs = list(_tc_layer(ags, xs, deg16, wls[layer], bls[layer], wrs[layer]))

    ags = _sc_agg(src_r, dst_r, *xs, z16)
    out = _tc_final(ags, xs, deg16, bidx2, wls[2], bls[2], wrs[2],
                    W0.T, b0.reshape(1, H), W1.T, b1.reshape(1, H),
                    W2.T, b2.reshape(1, H), W3.T, b3.reshape(1, 1))
    return out.reshape(B)
